# baseline (speedup 1.0000x reference)
import functools

import jax
import jax.numpy as jnp
from jax import lax
from jax.experimental import pallas as pl
from jax.experimental.pallas import tpu as pltpu
from jax.experimental.pallas import tpu_sc as plsc

_N = 1000000
_BATCH = 16384
_K = 32
_L = 16
_NC, _NS = 2, 16
_NW = _NC * _NS
_BPW = _BATCH // _NW
_BLKS = _BPW // _L

_BLK = 49152
_G = -(-_N // _BLK)


def _matvec_body(wt_ref, ht_ref, wu_ref, wv_ref, au_ref, av_ref):
    wu = wu_ref[...][:, 0:1]
    wv = wv_ref[...][:, 0:1]
    au_ref[...] = jnp.sum(wt_ref[...] * wu, axis=0)
    av_ref[...] = jnp.sum(ht_ref[...] * wv, axis=0)


_matvec = pl.pallas_call(
    _matvec_body,
    grid=(_G,),
    in_specs=[
        pl.BlockSpec((_K, _BLK), lambda i: (0, i)),
        pl.BlockSpec((_K, _BLK), lambda i: (0, i)),
        pl.BlockSpec((_K, 128), lambda i: (0, 0)),
        pl.BlockSpec((_K, 128), lambda i: (0, 0)),
    ],
    out_specs=[
        pl.BlockSpec((_BLK,), lambda i: (i,)),
        pl.BlockSpec((_BLK,), lambda i: (i,)),
    ],
    out_shape=[jax.ShapeDtypeStruct((_N,), jnp.float32)] * 2,
)

_mesh = plsc.VectorSubcoreMesh(core_axis_name="c", subcore_axis_name="s")


@functools.partial(
    pl.kernel,
    mesh=_mesh,
    out_type=jax.ShapeDtypeStruct((_BATCH,), jnp.float32),
    scratch_types=[
        pltpu.VMEM((2, _BPW), jnp.int32),
        pltpu.VMEM((_BPW,), jnp.int32),
        pltpu.VMEM((_BPW,), jnp.int32),
        pltpu.VMEM((_BPW,), jnp.float32),
        pltpu.VMEM((_BPW,), jnp.float32),
        pltpu.VMEM((_L,), jnp.float32),
        pltpu.VMEM((_BPW,), jnp.float32),
        pltpu.SemaphoreType.DMA,
        pltpu.SemaphoreType.DMA,
    ],
)
def _gather_sig(xt_hbm, au_hbm, av_hbm, wb_hbm, out_hbm,
                x2_v, uidx_v, vidx_v, au_v, av_v, wb_v, out_v, sem_u, sem_v):
    wid = lax.axis_index("s") * _NC + lax.axis_index("c")
    base = pl.multiple_of(wid * _BPW, 128)
    pltpu.sync_copy(xt_hbm.at[:, pl.ds(base, _BPW)], x2_v)
    pltpu.sync_copy(wb_hbm, wb_v)
    for blk in range(_BLKS):
        sl = pl.ds(blk * _L, _L)
        uidx_v[sl] = x2_v[0, sl]
        vidx_v[sl] = x2_v[1, sl]
    cu = pltpu.async_copy(au_hbm.at[uidx_v], au_v, sem_u)
    cv = pltpu.async_copy(av_hbm.at[vidx_v], av_v, sem_v)
    cu.wait()
    cv.wait()
    bias = wb_v[...]
    for blk in range(_BLKS):
        h = au_v[pl.ds(blk * _L, _L)] + av_v[pl.ds(blk * _L, _L)] + bias
        out_v[pl.ds(blk * _L, _L)] = 1.0 / (1.0 + jnp.exp(-h))
    pltpu.sync_copy(out_v, out_hbm.at[pl.ds(base, _BPW)])


def kernel(x, W, H, lin_w, lin_b):
    wu_b = jnp.broadcast_to(lin_w[0:_K], (_K, 128))
    wv_b = jnp.broadcast_to(lin_w[_K:2 * _K], (_K, 128))
    bias16 = jnp.broadcast_to(lin_b, (_L,))
    wt, bias16 = lax.optimization_barrier((W.T, bias16))
    au, av = _matvec(wt, H.T, wu_b, wv_b)
    return _gather_sig(x.T, au, av, bias16)


# speedup vs baseline: 1.0048x; 1.0048x over previous
import functools

import jax
import jax.numpy as jnp
from jax import lax
from jax.experimental import pallas as pl
from jax.experimental.pallas import tpu as pltpu
from jax.experimental.pallas import tpu_sc as plsc

_N = 1000000
_BATCH = 16384
_K = 32
_L = 16
_NC, _NS = 2, 16
_NW = _NC * _NS
_BPW = _BATCH // _NW
_BLKS = _BPW // _L

_BLK = 40960
_G = -(-_N // _BLK)


def _matvec_body(wt_ref, ht_ref, wu_ref, wv_ref, au_ref, av_ref):
    wu = wu_ref[...][:, 0:1]
    wv = wv_ref[...][:, 0:1]
    au_ref[...] = jnp.sum(wt_ref[...] * wu, axis=0)
    av_ref[...] = jnp.sum(ht_ref[...] * wv, axis=0)


_matvec = pl.pallas_call(
    _matvec_body,
    grid=(_G,),
    in_specs=[
        pl.BlockSpec((_K, _BLK), lambda i: (0, i)),
        pl.BlockSpec((_K, _BLK), lambda i: (0, i)),
        pl.BlockSpec((_K, 128), lambda i: (0, 0)),
        pl.BlockSpec((_K, 128), lambda i: (0, 0)),
    ],
    out_specs=[
        pl.BlockSpec((_BLK,), lambda i: (i,)),
        pl.BlockSpec((_BLK,), lambda i: (i,)),
    ],
    out_shape=[jax.ShapeDtypeStruct((_N,), jnp.float32)] * 2,
)

_mesh = plsc.VectorSubcoreMesh(core_axis_name="c", subcore_axis_name="s")


@functools.partial(
    pl.kernel,
    mesh=_mesh,
    out_type=jax.ShapeDtypeStruct((_BATCH,), jnp.float32),
    scratch_types=[
        pltpu.VMEM((2, _BPW), jnp.int32),
        pltpu.VMEM((_BPW,), jnp.int32),
        pltpu.VMEM((_BPW,), jnp.int32),
        pltpu.VMEM((_BPW,), jnp.float32),
        pltpu.VMEM((_BPW,), jnp.float32),
        pltpu.VMEM((_L,), jnp.float32),
        pltpu.VMEM((_BPW,), jnp.float32),
        pltpu.SemaphoreType.DMA,
        pltpu.SemaphoreType.DMA,
    ],
)
def _gather_sig(xt_hbm, au_hbm, av_hbm, wb_hbm, out_hbm,
                x2_v, uidx_v, vidx_v, au_v, av_v, wb_v, out_v, sem_u, sem_v):
    wid = lax.axis_index("s") * _NC + lax.axis_index("c")
    base = pl.multiple_of(wid * _BPW, 128)
    pltpu.sync_copy(xt_hbm.at[:, pl.ds(base, _BPW)], x2_v)
    pltpu.sync_copy(wb_hbm, wb_v)
    for blk in range(_BLKS):
        sl = pl.ds(blk * _L, _L)
        uidx_v[sl] = x2_v[0, sl]
        vidx_v[sl] = x2_v[1, sl]
    cu = pltpu.async_copy(au_hbm.at[uidx_v], au_v, sem_u)
    cv = pltpu.async_copy(av_hbm.at[vidx_v], av_v, sem_v)
    cu.wait()
    cv.wait()
    bias = wb_v[...]
    for blk in range(_BLKS):
        h = au_v[pl.ds(blk * _L, _L)] + av_v[pl.ds(blk * _L, _L)] + bias
        out_v[pl.ds(blk * _L, _L)] = 1.0 / (1.0 + jnp.exp(-h))
    pltpu.sync_copy(out_v, out_hbm.at[pl.ds(base, _BPW)])


def kernel(x, W, H, lin_w, lin_b):
    wu_b = jnp.broadcast_to(lin_w[0:_K], (_K, 128))
    wv_b = jnp.broadcast_to(lin_w[_K:2 * _K], (_K, 128))
    bias16 = jnp.broadcast_to(lin_b, (_L,))
    wt, bias16 = lax.optimization_barrier((W.T, bias16))
    au, av = _matvec(wt, H.T, wu_b, wv_b)
    return _gather_sig(x.T, au, av, bias16)
